# Initial kernel scaffold; baseline (speedup 1.0000x reference)
import jax
import jax.numpy as jnp
from jax.experimental import pallas as pl


def kernel(x_a, x_b, edge_index_b2a, edge_index_a2b, edge_index_a2a, projW_a, projb_a, projW_b, projb_b, asrc_b2a, adst_b2a, asrc_a2b, adst_a2b, asrc_a2a, adst_a2a, q, kW, kb, linW, linb):
    raise NotImplementedError("write your pallas kernel here")


# speedup vs baseline: 18.8375x; 18.8375x over previous
import jax
import jax.numpy as jnp
from jax import lax
from jax.experimental import pallas as pl
from jax.experimental.pallas import tpu as pltpu
from jax.experimental.pallas import tpu_sc as plsc

N_NODES = 10000
D_IN = 128
D_H = 64
D_PAD = 80
E_EDGES = 320000
BLK = 1000
N_BLK = N_NODES // BLK
CHUNK = 80
N_WORKERS = 32
EDGES_PER_TILE = E_EDGES // N_WORKERS
N_CHUNKS = EDGES_PER_TILE // CHUNK
N_PAD_NODES = 10240
ROWS_PER_TILE = 640


def _prologue_body(xa_ref, xb_ref, wa_ref, wb_ref, ba_ref, bb_ref,
                   aa_ref, ab_ref, ha_ref, hb_ref, s_ref, c_ref, smax_ref):
    i = pl.program_id(0)
    ha = jnp.dot(xa_ref[...], wa_ref[...],
                 preferred_element_type=jnp.float32) + ba_ref[...]
    hb = jnp.dot(xb_ref[...], wb_ref[...],
                 preferred_element_type=jnp.float32) + bb_ref[...]
    ha_ref[...] = ha
    hb_ref[...] = hb
    s = (jnp.dot(hb, ab_ref[...], preferred_element_type=jnp.float32)
         + jnp.dot(ha, aa_ref[...], preferred_element_type=jnp.float32))
    s_ref[...] = s
    blk_max = jnp.max(s, axis=0, keepdims=True)

    @pl.when(i == 0)
    def _():
        smax_ref[...] = blk_max

    @pl.when(i > 0)
    def _():
        smax_ref[...] = jnp.maximum(smax_ref[...], blk_max)

    @pl.when(i == N_BLK - 1)
    def _():
        m = smax_ref[...]
        w0 = m[:, 0:1] + m[:, 1:2]
        w1 = m[:, 2:3] + m[:, 3:4]
        c0 = jnp.where(w0 > 0, w0, 0.2 * w0)
        c1 = jnp.where(w1 > 0, w1, 0.2 * w1)
        c_ref[0:1, :] = jnp.broadcast_to(c0, (1, 16))
        c_ref[1:2, :] = jnp.broadcast_to(c1, (1, 16))


def _run_prologue(x_a, x_b, wa, wb, ba, bb, a_on_ha, a_on_hb):
    return pl.pallas_call(
        _prologue_body,
        grid=(N_BLK,),
        in_specs=[
            pl.BlockSpec((BLK, D_IN), lambda i: (i, 0)),
            pl.BlockSpec((BLK, D_IN), lambda i: (i, 0)),
            pl.BlockSpec((D_IN, D_H), lambda i: (0, 0)),
            pl.BlockSpec((D_IN, D_H), lambda i: (0, 0)),
            pl.BlockSpec((1, D_H), lambda i: (0, 0)),
            pl.BlockSpec((1, D_H), lambda i: (0, 0)),
            pl.BlockSpec((D_H, 8), lambda i: (0, 0)),
            pl.BlockSpec((D_H, 8), lambda i: (0, 0)),
        ],
        out_specs=[
            pl.BlockSpec((BLK, D_H), lambda i: (i, 0)),
            pl.BlockSpec((BLK, D_H), lambda i: (i, 0)),
            pl.BlockSpec((BLK, 8), lambda i: (i, 0)),
            pl.BlockSpec((2, 16), lambda i: (0, 0)),
        ],
        out_shape=[
            jax.ShapeDtypeStruct((N_NODES, D_H), jnp.float32),
            jax.ShapeDtypeStruct((N_NODES, D_H), jnp.float32),
            jax.ShapeDtypeStruct((N_NODES, 8), jnp.float32),
            jax.ShapeDtypeStruct((2, 16), jnp.float32),
        ],
        scratch_shapes=[pltpu.VMEM((1, 8), jnp.float32)],
    )(x_a, x_b, wa, wb, ba, bb, a_on_ha, a_on_hb)


def _make_sc_body(r):
    def _sc_body(esrc, edst, h_hbm, sarr, cvec, accs,
                 s_v, rows_v, stg_v, sidx_v, didx_v, ex_v, c_v, acc_sh,
                 sem):
        cid = lax.axis_index("c")
        sid = lax.axis_index("s")
        wid = cid * 16 + sid
        onehot0 = jnp.where(lax.iota(jnp.int32, 16) == 0,
                            jnp.float32(1), jnp.float32(0))

        pltpu.sync_copy(sarr, s_v)
        pltpu.sync_copy(cvec, c_v)
        cshift = c_v[pl.ds(r * 16, 16)]

        def _zrow(i, carry):
            for j5 in range(D_PAD // 16):
                stg_v[i, pl.ds(j5 * 16, 16)] = jnp.zeros((16,), jnp.float32)
            return carry

        lax.fori_loop(0, CHUNK, _zrow, 0)
        base = sid * ROWS_PER_TILE
        for i in range(ROWS_PER_TILE // CHUNK):
            pltpu.sync_copy(stg_v, acc_sh.at[pl.ds(base + i * CHUNK, CHUNK)])
        plsc.subcore_barrier()

        def _chunk(c, carry):
            off = wid * EDGES_PER_TILE + c * CHUNK
            pltpu.sync_copy(esrc.at[pl.ds(off, CHUNK)], sidx_v)
            pltpu.sync_copy(edst.at[pl.ds(off, CHUNK)], didx_v)
            pltpu.async_copy(h_hbm.at[sidx_v], rows_v, sem).wait()

            def _ex(k, carry2):
                sv = sidx_v[pl.ds(k * 16, 16)]
                dv = didx_v[pl.ds(k * 16, 16)]
                sa = plsc.load_gather(s_v, [sv * 2])
                sd = plsc.load_gather(s_v, [dv * 2 + 1])
                al = sa + sd
                al = jnp.where(al > 0, al, 0.2 * al)
                ex_v[pl.ds(k * 16, 16)] = jnp.exp(al - cshift)
                return carry2

            lax.fori_loop(0, CHUNK // 16, _ex, 0)

            def _scale(e, carry2):
                ev = plsc.load_gather(ex_v, [jnp.full((16,), e, jnp.int32)])
                for j in range(D_H // 16):
                    stg_v[e, pl.ds(j * 16, 16)] = (
                        rows_v[e, pl.ds(j * 16, 16)] * ev)
                stg_v[e, pl.ds(D_H, 16)] = ev * onehot0
                return carry2

            lax.fori_loop(0, CHUNK, _scale, 0)
            pltpu.sync_copy(stg_v, acc_sh.at[didx_v], add=True)
            return carry

        lax.fori_loop(0, N_CHUNKS, _chunk, 0)
        plsc.subcore_barrier()
        pltpu.sync_copy(acc_sh.at[pl.ds(base, ROWS_PER_TILE)],
                        accs.at[cid, pl.ds(base, ROWS_PER_TILE)])

    return _sc_body


def _run_sc(r, esrc, edst, h, sarr, cvec):
    mesh = plsc.VectorSubcoreMesh(core_axis_name="c", subcore_axis_name="s")
    f = pl.kernel(
        _make_sc_body(r),
        mesh=mesh,
        compiler_params=pltpu.CompilerParams(needs_layout_passes=False,
                                             use_tc_tiling_on_sc=False),
        out_type=jax.ShapeDtypeStruct((2, N_PAD_NODES, D_PAD), jnp.float32),
        scratch_types=[
            pltpu.VMEM((N_NODES * 2,), jnp.float32),
            pltpu.VMEM((CHUNK, D_H), jnp.float32),
            pltpu.VMEM((CHUNK, D_PAD), jnp.float32),
            pltpu.VMEM((CHUNK,), jnp.int32),
            pltpu.VMEM((CHUNK,), jnp.int32),
            pltpu.VMEM((CHUNK,), jnp.float32),
            pltpu.VMEM((32,), jnp.float32),
            pltpu.VMEM_SHARED((N_PAD_NODES, D_PAD), jnp.float32),
            pltpu.SemaphoreType.DMA,
        ],
    )
    return f(esrc, edst, h, sarr, cvec)


def _epilogue_body(acc0_ref, acc1_ref, kw_ref, kb_ref, q_ref, lw_ref,
                   lb_ref, y_ref, sums_ref, attn_ref, mx_ref):
    p = pl.program_id(0)
    b = pl.program_id(1)

    a00 = acc0_ref[0]
    a01 = acc0_ref[1]
    a10 = acc1_ref[0]
    a11 = acc1_ref[1]
    s0 = a00 + a01
    s1 = a10 + a11
    o0 = jax.nn.relu(s0[:, :D_H] / (s0[:, D_H:D_H + 1] + 1e-16))
    o1 = jax.nn.relu(s1[:, :D_H] / (s1[:, D_H:D_H + 1] + 1e-16))

    @pl.when(p == 0)
    def _():
        t0 = jnp.tanh(jnp.dot(o0, kw_ref[...],
                              preferred_element_type=jnp.float32)
                      + kb_ref[...])
        t1 = jnp.tanh(jnp.dot(o1, kw_ref[...],
                              preferred_element_type=jnp.float32)
                      + kb_ref[...])
        part = jnp.concatenate(
            [jnp.sum(t0, axis=0, keepdims=True),
             jnp.sum(t1, axis=0, keepdims=True)], axis=0)

        @pl.when(b == 0)
        def _():
            sums_ref[...] = part

        @pl.when(b > 0)
        def _():
            sums_ref[...] = sums_ref[...] + part

        @pl.when(b == N_BLK - 1)
        def _():
            m = sums_ref[...] / float(N_NODES)
            sc = jnp.sum(q_ref[...] * m, axis=1, keepdims=True)
            smax = jnp.max(sc, axis=0, keepdims=True)
            e = jnp.exp(sc - smax)
            attn = e / jnp.sum(e, axis=0, keepdims=True)
            attn_ref[...] = jnp.broadcast_to(attn, (2, D_H))

    @pl.when(p == 1)
    def _():
        w = attn_ref[0:1, :] * o0 + attn_ref[1:2, :] * o1
        blk_mx = jnp.max(w, axis=0, keepdims=True)

        @pl.when(b == 0)
        def _():
            mx_ref[...] = blk_mx

        @pl.when(b > 0)
        def _():
            mx_ref[...] = jnp.maximum(mx_ref[...], blk_mx)

        @pl.when(b == N_BLK - 1)
        def _():
            y_ref[...] = (jnp.dot(mx_ref[...], lw_ref[...],
                                  preferred_element_type=jnp.float32)
                          + lb_ref[...])


def _run_epilogue(acc0, acc1, kW, kb2, q2, linW, linb2):
    return pl.pallas_call(
        _epilogue_body,
        grid=(2, N_BLK),
        in_specs=[
            pl.BlockSpec((2, BLK, D_PAD), lambda p, b: (0, b, 0)),
            pl.BlockSpec((2, BLK, D_PAD), lambda p, b: (0, b, 0)),
            pl.BlockSpec((D_H, D_H), lambda p, b: (0, 0)),
            pl.BlockSpec((1, D_H), lambda p, b: (0, 0)),
            pl.BlockSpec((1, D_H), lambda p, b: (0, 0)),
            pl.BlockSpec((D_H, 1), lambda p, b: (0, 0)),
            pl.BlockSpec((1, 1), lambda p, b: (0, 0)),
        ],
        out_specs=pl.BlockSpec((1, 1), lambda p, b: (0, 0)),
        out_shape=jax.ShapeDtypeStruct((1, 1), jnp.float32),
        scratch_shapes=[
            pltpu.VMEM((2, D_H), jnp.float32),
            pltpu.VMEM((2, D_H), jnp.float32),
            pltpu.VMEM((1, D_H), jnp.float32),
        ],
    )(acc0, acc1, kW, kb2, q2, linW, linb2)


def kernel(x_a, x_b, edge_index_b2a, edge_index_a2b, edge_index_a2a,
           projW_a, projb_a, projW_b, projb_b,
           asrc_b2a, adst_b2a, asrc_a2b, adst_a2b, asrc_a2a, adst_a2a,
           q, kW, kb, linW, linb):
    del edge_index_a2b, asrc_a2b, adst_a2b
    zeros_h = jnp.zeros((D_H,), jnp.float32)
    a_on_ha = jnp.stack([zeros_h, adst_b2a, asrc_a2a, adst_a2a,
                         zeros_h, zeros_h, zeros_h, zeros_h], axis=1)
    a_on_hb = jnp.stack([asrc_b2a, zeros_h, zeros_h, zeros_h,
                         zeros_h, zeros_h, zeros_h, zeros_h], axis=1)
    ha, hb, sarr, cvec = _run_prologue(
        x_a, x_b, projW_a, projW_b,
        projb_a.reshape(1, D_H), projb_b.reshape(1, D_H), a_on_ha, a_on_hb)

    e0 = edge_index_b2a.astype(jnp.int32)
    e1 = edge_index_a2a.astype(jnp.int32)
    c_flat = cvec.reshape(-1)
    s0 = sarr[:, 0:2].reshape(-1)
    s1 = sarr[:, 2:4].reshape(-1)
    acc0 = _run_sc(0, e0[0], e0[1], hb, s0, c_flat)
    acc1 = _run_sc(1, e1[0], e1[1], ha, s1, c_flat)

    return _run_epilogue(acc0, acc1, kW, kb.reshape(1, D_H),
                         q.reshape(1, D_H), linW, linb.reshape(1, 1))
